# baseline (speedup 1.0000x reference)
import functools

import jax
import jax.numpy as jnp
from jax import lax
from jax.experimental import pallas as pl
from jax.experimental.pallas import tpu as pltpu
from jax.experimental.pallas import tpu_sc as plsc

N = 10000
D = 128
E = 320000

NC = 2
NS = 16
NW = NC * NS

EPW = E // NW
BLK = 80
NBLK = EPW // BLK

KDEG = 5
RCHUNK = 400
NRCHUNK = N // RCHUNK

ROWS_TC = 1000
GRID_TC = N // ROWS_TC

_mesh = plsc.VectorSubcoreMesh(core_axis_name="c", subcore_axis_name="s")


@functools.partial(
    pl.kernel,
    mesh=_mesh,
    out_type=jax.ShapeDtypeStruct((NC, N, D), jnp.float32),
    scratch_types=[
        pltpu.VMEM((NBLK, BLK), jnp.int32),
        pltpu.VMEM((BLK, D), jnp.float32),
        pltpu.VMEM_SHARED((N, D), jnp.float32),
        pltpu.SemaphoreType.DMA,
    ],
)
def _deg_kernel(dst_hbm, ones_hbm, zeros_hbm, out_hbm, dst_v, ones_v, acc, sem):
    cid = lax.axis_index("c")
    sid = lax.axis_index("s")
    wid = sid * NC + cid

    @pl.loop(0, NRCHUNK)
    def _(ch):
        @pl.when(ch % NS == sid)
        def _():
            pltpu.sync_copy(zeros_hbm.at[pl.ds(ch * RCHUNK, RCHUNK)],
                            acc.at[pl.ds(ch * RCHUNK, RCHUNK)])

    pltpu.sync_copy(dst_hbm.at[wid], dst_v)
    pltpu.sync_copy(ones_hbm, ones_v)
    plsc.subcore_barrier()

    @pl.loop(0, NBLK // KDEG)
    def _(jb):
        for k in range(KDEG):
            pltpu.async_copy(ones_v, acc.at[dst_v.at[jb * KDEG + k]],
                             sem, add=True)
        for k in range(KDEG):
            pltpu.make_async_copy(ones_v, acc.at[dst_v.at[jb * KDEG + k]],
                                  sem).wait()
    for j in range((NBLK // KDEG) * KDEG, NBLK):
        pltpu.sync_copy(ones_v, acc.at[dst_v.at[j]], add=True)

    plsc.subcore_barrier()

    @pl.loop(0, NRCHUNK)
    def _(ch):
        @pl.when(ch % NS == sid)
        def _():
            pltpu.sync_copy(acc.at[pl.ds(ch * RCHUNK, RCHUNK)],
                            out_hbm.at[cid, pl.ds(ch * RCHUNK, RCHUNK)])


NBUF = 2
NFULL = (NBLK // NBUF) * NBUF


@functools.partial(
    pl.kernel,
    mesh=_mesh,
    out_type=jax.ShapeDtypeStruct((NC, N, D), jnp.float32),
    scratch_types=[
        pltpu.VMEM((EPW,), jnp.int32),
        pltpu.VMEM((NBLK, BLK), jnp.int32),
        pltpu.VMEM((BLK, D), jnp.float32),
        pltpu.VMEM((BLK, D), jnp.float32),
        pltpu.VMEM_SHARED((N, D), jnp.float32),
        pltpu.SemaphoreType.DMA,
        pltpu.SemaphoreType.DMA,
        pltpu.SemaphoreType.DMA,
        pltpu.SemaphoreType.DMA,
    ],
)
def _agg_kernel(y_hbm, src_hbm, dst_hbm, zeros_hbm, out_hbm,
                src_v, dst_v, rows0, rows1, acc, gsem0, gsem1, ssem0, ssem1):
    cid = lax.axis_index("c")
    sid = lax.axis_index("s")
    wid = sid * NC + cid
    rows = [rows0, rows1]
    gsems = [gsem0, gsem1]
    ssems = [ssem0, ssem1]

    @pl.loop(0, NRCHUNK)
    def _(ch):
        @pl.when(ch % NS == sid)
        def _():
            pltpu.sync_copy(zeros_hbm.at[pl.ds(ch * RCHUNK, RCHUNK)],
                            acc.at[pl.ds(ch * RCHUNK, RCHUNK)])

    pltpu.sync_copy(src_hbm.at[wid], src_v)
    pltpu.sync_copy(dst_hbm.at[wid], dst_v)
    plsc.subcore_barrier()

    def gather(j, b):
        pltpu.async_copy(y_hbm.at[src_v.at[pl.ds(j * BLK, BLK)]],
                         rows[b], gsems[b])

    def wait_gather(j, b):
        pltpu.make_async_copy(y_hbm.at[src_v.at[pl.ds(j * BLK, BLK)]],
                              rows[b], gsems[b]).wait()

    def scatter(j, b):
        pltpu.async_copy(rows[b], acc.at[dst_v.at[j]], ssems[b], add=True)

    def wait_scatter(j, b):
        pltpu.make_async_copy(rows[b], acc.at[dst_v.at[j]], ssems[b]).wait()

    gather(0, 0)
    gather(1, 1)
    wait_gather(0, 0)
    scatter(0, 0)

    @pl.loop(1, NBLK, step=2)
    def _(g):
        wait_gather(g, 1)
        scatter(g, 1)
        wait_scatter(g - 1, 0)
        gather(g + 1, 0)
        wait_gather(g + 1, 0)
        scatter(g + 1, 0)
        wait_scatter(g, 1)

        @pl.when(g + 2 < NBLK)
        def _():
            gather(g + 2, 1)

    wait_scatter(NBLK - 1, 0)
    plsc.subcore_barrier()

    @pl.loop(0, NRCHUNK)
    def _(ch):
        @pl.when(ch % NS == sid)
        def _():
            pltpu.sync_copy(acc.at[pl.ds(ch * RCHUNK, RCHUNK)],
                            out_hbm.at[cid, pl.ds(ch * RCHUNK, RCHUNK)])


def _matmul_body(x_ref, w_ref, o_ref):
    o_ref[...] = jnp.dot(x_ref[...], w_ref[...],
                         preferred_element_type=jnp.float32)


def _matmul(x, w):
    return pl.pallas_call(
        _matmul_body,
        grid=(GRID_TC,),
        in_specs=[
            pl.BlockSpec((ROWS_TC, D), lambda i: (i, 0)),
            pl.BlockSpec((D, D), lambda i: (0, 0)),
        ],
        out_specs=pl.BlockSpec((ROWS_TC, D), lambda i: (i, 0)),
        out_shape=jax.ShapeDtypeStruct((N, D), jnp.float32),
    )(x, w)


def _scale_body(degp_ref, xw_ref, y_ref, dis_ref):
    deg = degp_ref[0, :, 0:1] + degp_ref[1, :, 0:1]
    dis = lax.rsqrt(deg + 1.0)
    y_ref[...] = xw_ref[...] * dis
    dis_ref[...] = jnp.broadcast_to(dis, dis_ref.shape)


def _scale(deg_partials, xw):
    return pl.pallas_call(
        _scale_body,
        grid=(GRID_TC,),
        in_specs=[
            pl.BlockSpec((NC, ROWS_TC, D), lambda i: (0, i, 0)),
            pl.BlockSpec((ROWS_TC, D), lambda i: (i, 0)),
        ],
        out_specs=[
            pl.BlockSpec((ROWS_TC, D), lambda i: (i, 0)),
            pl.BlockSpec((ROWS_TC, 16), lambda i: (i, 0)),
        ],
        out_shape=[
            jax.ShapeDtypeStruct((N, D), jnp.float32),
            jax.ShapeDtypeStruct((N, 16), jnp.float32),
        ],
    )(deg_partials, xw)


def _mid_body(pa_ref, y_ref, dis_ref, b_ref, w_ref, y2_ref):
    agg = pa_ref[0] + pa_ref[1] + y_ref[...]
    dis = dis_ref[:, 0:1]
    h = jnp.maximum(agg * dis + b_ref[...], 0.0)
    y2_ref[...] = jnp.dot(h, w_ref[...],
                          preferred_element_type=jnp.float32) * dis


def _mid(partials, y, dis16, b, w):
    return pl.pallas_call(
        _mid_body,
        grid=(GRID_TC,),
        in_specs=[
            pl.BlockSpec((NC, ROWS_TC, D), lambda i: (0, i, 0)),
            pl.BlockSpec((ROWS_TC, D), lambda i: (i, 0)),
            pl.BlockSpec((ROWS_TC, 16), lambda i: (i, 0)),
            pl.BlockSpec((1, D), lambda i: (0, 0)),
            pl.BlockSpec((D, D), lambda i: (0, 0)),
        ],
        out_specs=pl.BlockSpec((ROWS_TC, D), lambda i: (i, 0)),
        out_shape=jax.ShapeDtypeStruct((N, D), jnp.float32),
    )(partials, y, dis16, b, w)


def _final_body(pa_ref, y_ref, dis_ref, b_ref, o_ref):
    agg = pa_ref[0] + pa_ref[1] + y_ref[...]
    o_ref[...] = agg * dis_ref[:, 0:1] + b_ref[...]


def _final(partials, y, dis16, b):
    return pl.pallas_call(
        _final_body,
        grid=(GRID_TC,),
        in_specs=[
            pl.BlockSpec((NC, ROWS_TC, D), lambda i: (0, i, 0)),
            pl.BlockSpec((ROWS_TC, D), lambda i: (i, 0)),
            pl.BlockSpec((ROWS_TC, 16), lambda i: (i, 0)),
            pl.BlockSpec((1, D), lambda i: (0, 0)),
        ],
        out_specs=pl.BlockSpec((ROWS_TC, D), lambda i: (i, 0)),
        out_shape=jax.ShapeDtypeStruct((N, D), jnp.float32),
    )(partials, y, dis16, b)


def kernel(x, edge_index, W1, b1, W2, b2):
    srcA = edge_index[0].astype(jnp.int32).reshape(NW, EPW)
    dst = edge_index[1].astype(jnp.int32).reshape(NW, NBLK, BLK)
    ones128 = jnp.ones((BLK, D), jnp.float32)
    zeros128 = jnp.zeros((N, D), jnp.float32)
    b1r = b1.reshape(1, D)
    b2r = b2.reshape(1, D)

    deg_partials = _deg_kernel(dst, ones128, zeros128)
    xw1 = _matmul(x, W1)
    y1, dis16 = _scale(deg_partials, xw1)
    p1 = _agg_kernel(y1, srcA, dst, zeros128)
    y2 = _mid(p1, y1, dis16, b1r, W2)
    p2 = _agg_kernel(y2, srcA, dst, zeros128)
    return _final(p2, y2, dis16, b2r)


# speedup vs baseline: 1.1939x; 1.1939x over previous
import functools

import jax
import jax.numpy as jnp
from jax import lax
from jax.experimental import pallas as pl
from jax.experimental.pallas import tpu as pltpu
from jax.experimental.pallas import tpu_sc as plsc

N = 10000
D = 128
E = 320000

NC = 2
NS = 16
NW = NC * NS

EPW = E // NW
BLK = 80
NBLK = EPW // BLK

KDEG = 5
RCHUNK = 400
NRCHUNK = N // RCHUNK

ROWS_TC = 1000
GRID_TC = N // ROWS_TC

_mesh = plsc.VectorSubcoreMesh(core_axis_name="c", subcore_axis_name="s")


@functools.partial(
    pl.kernel,
    mesh=_mesh,
    out_type=jax.ShapeDtypeStruct((NC, N, D), jnp.float32),
    scratch_types=[
        pltpu.VMEM((NBLK, BLK), jnp.int32),
        pltpu.VMEM((BLK, D), jnp.float32),
        pltpu.VMEM_SHARED((N, D), jnp.float32),
        pltpu.SemaphoreType.DMA,
    ],
)
def _deg_kernel(dst_hbm, ones_hbm, zeros_hbm, out_hbm, dst_v, ones_v, acc, sem):
    cid = lax.axis_index("c")
    sid = lax.axis_index("s")
    wid = sid * NC + cid

    @pl.loop(0, NRCHUNK)
    def _(ch):
        @pl.when(ch % NS == sid)
        def _():
            pltpu.sync_copy(zeros_hbm.at[pl.ds(ch * RCHUNK, RCHUNK)],
                            acc.at[pl.ds(ch * RCHUNK, RCHUNK)])

    pltpu.sync_copy(dst_hbm.at[wid], dst_v)
    pltpu.sync_copy(ones_hbm, ones_v)
    plsc.subcore_barrier()

    @pl.loop(0, NBLK // KDEG)
    def _(jb):
        for k in range(KDEG):
            pltpu.async_copy(ones_v, acc.at[dst_v.at[jb * KDEG + k]],
                             sem, add=True)
        for k in range(KDEG):
            pltpu.make_async_copy(ones_v, acc.at[dst_v.at[jb * KDEG + k]],
                                  sem).wait()
    for j in range((NBLK // KDEG) * KDEG, NBLK):
        pltpu.sync_copy(ones_v, acc.at[dst_v.at[j]], add=True)

    plsc.subcore_barrier()

    @pl.loop(0, NRCHUNK)
    def _(ch):
        @pl.when(ch % NS == sid)
        def _():
            pltpu.sync_copy(acc.at[pl.ds(ch * RCHUNK, RCHUNK)],
                            out_hbm.at[cid, pl.ds(ch * RCHUNK, RCHUNK)])


NBUF = 2
NFULL = (NBLK // NBUF) * NBUF


@functools.partial(
    pl.kernel,
    mesh=_mesh,
    out_type=jax.ShapeDtypeStruct((NC, N, D), jnp.float32),
    scratch_types=[
        pltpu.VMEM((EPW,), jnp.int32),
        pltpu.VMEM((NBLK, BLK), jnp.int32),
        pltpu.VMEM((BLK, D), jnp.float32),
        pltpu.VMEM((BLK, D), jnp.float32),
        pltpu.VMEM_SHARED((N, D), jnp.float32),
        pltpu.SemaphoreType.DMA,
        pltpu.SemaphoreType.DMA,
    ],
)
def _agg_kernel(y_hbm, src_hbm, dst_hbm, zeros_hbm, out_hbm,
                src_v, dst_v, rows0, rows1, acc, sem0, sem1):
    cid = lax.axis_index("c")
    sid = lax.axis_index("s")
    wid = sid * NC + cid
    rows = [rows0, rows1]
    sems = [sem0, sem1]

    @pl.loop(0, NRCHUNK)
    def _(ch):
        @pl.when(ch % NS == sid)
        def _():
            pltpu.sync_copy(zeros_hbm.at[pl.ds(ch * RCHUNK, RCHUNK)],
                            acc.at[pl.ds(ch * RCHUNK, RCHUNK)])

    pltpu.sync_copy(src_hbm.at[wid], src_v)
    pltpu.sync_copy(dst_hbm.at[wid], dst_v)
    plsc.subcore_barrier()

    for b in range(NBUF):
        pltpu.async_copy(y_hbm.at[src_v.at[pl.ds(b * BLK, BLK)]],
                         rows[b], sems[b])

    @pl.loop(0, NFULL, step=NBUF)
    def _(g):
        for b in range(NBUF):
            j = g + b
            pltpu.make_async_copy(y_hbm.at[src_v.at[pl.ds(j * BLK, BLK)]],
                                  rows[b], sems[b]).wait()
            pltpu.sync_copy(rows[b], acc.at[dst_v.at[j]], add=True)

            @pl.when(j + NBUF < NBLK)
            def _():
                pltpu.async_copy(
                    y_hbm.at[src_v.at[pl.ds((j + NBUF) * BLK, BLK)]],
                    rows[b], sems[b])

    for j in range(NFULL, NBLK):
        b = j % NBUF
        pltpu.make_async_copy(y_hbm.at[src_v.at[pl.ds(j * BLK, BLK)]],
                              rows[b], sems[b]).wait()
        pltpu.sync_copy(rows[b], acc.at[dst_v.at[j]], add=True)

    plsc.subcore_barrier()

    @pl.loop(0, NRCHUNK)
    def _(ch):
        @pl.when(ch % NS == sid)
        def _():
            pltpu.sync_copy(acc.at[pl.ds(ch * RCHUNK, RCHUNK)],
                            out_hbm.at[cid, pl.ds(ch * RCHUNK, RCHUNK)])


def _matmul_body(x_ref, w_ref, o_ref):
    o_ref[...] = jnp.dot(x_ref[...], w_ref[...],
                         preferred_element_type=jnp.float32)


def _matmul(x, w):
    return pl.pallas_call(
        _matmul_body,
        grid=(GRID_TC,),
        in_specs=[
            pl.BlockSpec((ROWS_TC, D), lambda i: (i, 0)),
            pl.BlockSpec((D, D), lambda i: (0, 0)),
        ],
        out_specs=pl.BlockSpec((ROWS_TC, D), lambda i: (i, 0)),
        out_shape=jax.ShapeDtypeStruct((N, D), jnp.float32),
    )(x, w)


def _scale_body(degp_ref, xw_ref, y_ref, dis_ref):
    deg = degp_ref[0, :, 0:1] + degp_ref[1, :, 0:1]
    dis = lax.rsqrt(deg + 1.0)
    y_ref[...] = xw_ref[...] * dis
    dis_ref[...] = jnp.broadcast_to(dis, dis_ref.shape)


def _scale(deg_partials, xw):
    return pl.pallas_call(
        _scale_body,
        grid=(GRID_TC,),
        in_specs=[
            pl.BlockSpec((NC, ROWS_TC, D), lambda i: (0, i, 0)),
            pl.BlockSpec((ROWS_TC, D), lambda i: (i, 0)),
        ],
        out_specs=[
            pl.BlockSpec((ROWS_TC, D), lambda i: (i, 0)),
            pl.BlockSpec((ROWS_TC, 16), lambda i: (i, 0)),
        ],
        out_shape=[
            jax.ShapeDtypeStruct((N, D), jnp.float32),
            jax.ShapeDtypeStruct((N, 16), jnp.float32),
        ],
    )(deg_partials, xw)


def _mid_body(pa_ref, y_ref, dis_ref, b_ref, w_ref, y2_ref):
    agg = pa_ref[0] + pa_ref[1] + y_ref[...]
    dis = dis_ref[:, 0:1]
    h = jnp.maximum(agg * dis + b_ref[...], 0.0)
    y2_ref[...] = jnp.dot(h, w_ref[...],
                          preferred_element_type=jnp.float32) * dis


def _mid(partials, y, dis16, b, w):
    return pl.pallas_call(
        _mid_body,
        grid=(GRID_TC,),
        in_specs=[
            pl.BlockSpec((NC, ROWS_TC, D), lambda i: (0, i, 0)),
            pl.BlockSpec((ROWS_TC, D), lambda i: (i, 0)),
            pl.BlockSpec((ROWS_TC, 16), lambda i: (i, 0)),
            pl.BlockSpec((1, D), lambda i: (0, 0)),
            pl.BlockSpec((D, D), lambda i: (0, 0)),
        ],
        out_specs=pl.BlockSpec((ROWS_TC, D), lambda i: (i, 0)),
        out_shape=jax.ShapeDtypeStruct((N, D), jnp.float32),
    )(partials, y, dis16, b, w)


def _final_body(pa_ref, y_ref, dis_ref, b_ref, o_ref):
    agg = pa_ref[0] + pa_ref[1] + y_ref[...]
    o_ref[...] = agg * dis_ref[:, 0:1] + b_ref[...]


def _final(partials, y, dis16, b):
    return pl.pallas_call(
        _final_body,
        grid=(GRID_TC,),
        in_specs=[
            pl.BlockSpec((NC, ROWS_TC, D), lambda i: (0, i, 0)),
            pl.BlockSpec((ROWS_TC, D), lambda i: (i, 0)),
            pl.BlockSpec((ROWS_TC, 16), lambda i: (i, 0)),
            pl.BlockSpec((1, D), lambda i: (0, 0)),
        ],
        out_specs=pl.BlockSpec((ROWS_TC, D), lambda i: (i, 0)),
        out_shape=jax.ShapeDtypeStruct((N, D), jnp.float32),
    )(partials, y, dis16, b)


def kernel(x, edge_index, W1, b1, W2, b2):
    srcA = edge_index[0].astype(jnp.int32).reshape(NW, EPW)
    dst = edge_index[1].astype(jnp.int32).reshape(NW, NBLK, BLK)
    ones128 = jnp.ones((BLK, D), jnp.float32)
    zeros128 = jnp.zeros((N, D), jnp.float32)
    b1r = b1.reshape(1, D)
    b2r = b2.reshape(1, D)

    deg_partials = _deg_kernel(dst, ones128, zeros128)
    xw1 = _matmul(x, W1)
    y1, dis16 = _scale(deg_partials, xw1)
    p1 = _agg_kernel(y1, srcA, dst, zeros128)
    y2 = _mid(p1, y1, dis16, b1r, W2)
    p2 = _agg_kernel(y2, srcA, dst, zeros128)
    return _final(p2, y2, dis16, b2r)


# speedup vs baseline: 1.2670x; 1.0613x over previous
import functools

import jax
import jax.numpy as jnp
from jax import lax
from jax.experimental import pallas as pl
from jax.experimental.pallas import tpu as pltpu
from jax.experimental.pallas import tpu_sc as plsc

N = 10000
D = 128
E = 320000

NC = 2
NS = 16
NW = NC * NS

EPW = E // NW
BLK = 125
NBLK = EPW // BLK
HBLK = NBLK // 2

KDEG = 5
RCHUNK = 400
NRCHUNK = N // RCHUNK

ROWS_TC = 1000
GRID_TC = N // ROWS_TC

_mesh = plsc.VectorSubcoreMesh(core_axis_name="c", subcore_axis_name="s")


@functools.partial(
    pl.kernel,
    mesh=_mesh,
    out_type=jax.ShapeDtypeStruct((NC, N, D), jnp.float32),
    scratch_types=[
        pltpu.VMEM((NBLK, BLK), jnp.int32),
        pltpu.VMEM((BLK, D), jnp.float32),
        pltpu.VMEM_SHARED((N, D), jnp.float32),
        pltpu.SemaphoreType.DMA,
    ],
)
def _deg_kernel(dst_hbm, ones_hbm, zeros_hbm, out_hbm, dst_v, ones_v, acc, sem):
    cid = lax.axis_index("c")
    sid = lax.axis_index("s")
    wid = sid * NC + cid

    @pl.loop(0, NRCHUNK)
    def _(ch):
        @pl.when(ch % NS == sid)
        def _():
            pltpu.sync_copy(zeros_hbm.at[pl.ds(ch * RCHUNK, RCHUNK)],
                            acc.at[pl.ds(ch * RCHUNK, RCHUNK)])

    pltpu.sync_copy(dst_hbm.at[wid], dst_v)
    pltpu.sync_copy(ones_hbm, ones_v)
    plsc.subcore_barrier()

    @pl.loop(0, NBLK // KDEG)
    def _(jb):
        for k in range(KDEG):
            pltpu.async_copy(ones_v, acc.at[dst_v.at[jb * KDEG + k]],
                             sem, add=True)
        for k in range(KDEG):
            pltpu.make_async_copy(ones_v, acc.at[dst_v.at[jb * KDEG + k]],
                                  sem).wait()
    for j in range((NBLK // KDEG) * KDEG, NBLK):
        pltpu.sync_copy(ones_v, acc.at[dst_v.at[j]], add=True)

    plsc.subcore_barrier()

    @pl.loop(0, NRCHUNK)
    def _(ch):
        @pl.when(ch % NS == sid)
        def _():
            pltpu.sync_copy(acc.at[pl.ds(ch * RCHUNK, RCHUNK)],
                            out_hbm.at[cid, pl.ds(ch * RCHUNK, RCHUNK)])


NBUF = 2
NFULL = (NBLK // NBUF) * NBUF


@functools.partial(
    pl.kernel,
    mesh=_mesh,
    out_type=jax.ShapeDtypeStruct((NC, N, D), jnp.float32),
    scratch_types=[
        pltpu.VMEM((NBLK, BLK), jnp.int32),
        pltpu.VMEM((HBLK, BLK), jnp.int32),
        pltpu.VMEM((BLK, D), jnp.float32),
        pltpu.VMEM((BLK, D), jnp.float32),
        pltpu.VMEM_SHARED((N, D), jnp.float32),
        pltpu.SemaphoreType.DMA,
        pltpu.SemaphoreType.DMA,
    ],
)
def _agg_kernel(y_hbm, src_hbm, dst_hbm, zeros_hbm, out_hbm,
                src_v, dst_v, rows0, rows1, acc, sem0, sem1):
    cid = lax.axis_index("c")
    sid = lax.axis_index("s")
    wid = sid * NC + cid
    rows = [rows0, rows1]
    sems = [sem0, sem1]

    @pl.loop(0, NRCHUNK)
    def _(ch):
        @pl.when(ch % NS == sid)
        def _():
            pltpu.sync_copy(zeros_hbm.at[pl.ds(ch * RCHUNK, RCHUNK)],
                            acc.at[pl.ds(ch * RCHUNK, RCHUNK)])

    pltpu.sync_copy(src_hbm.at[wid], src_v)
    pltpu.sync_copy(dst_hbm.at[wid, pl.ds(0, HBLK)], dst_v)
    plsc.subcore_barrier()

    for b in range(NBUF):
        pltpu.async_copy(y_hbm.at[src_v.at[b]], rows[b], sems[b])

    @pl.loop(0, NFULL, step=NBUF)
    def _(g):
        for b in range(NBUF):
            j = g + b
            if b == 0:
                @pl.when(g == HBLK)
                def _():
                    pltpu.sync_copy(dst_hbm.at[wid, pl.ds(HBLK, HBLK)], dst_v)
            pltpu.make_async_copy(y_hbm.at[src_v.at[j]],
                                  rows[b], sems[b]).wait()
            pltpu.sync_copy(rows[b], acc.at[dst_v.at[j % HBLK]], add=True)

            @pl.when(j + NBUF < NBLK)
            def _():
                pltpu.async_copy(y_hbm.at[src_v.at[j + NBUF]],
                                 rows[b], sems[b])

    for j in range(NFULL, NBLK):
        b = j % NBUF
        pltpu.make_async_copy(y_hbm.at[src_v.at[j]], rows[b], sems[b]).wait()
        pltpu.sync_copy(rows[b], acc.at[dst_v.at[j % HBLK]], add=True)

    plsc.subcore_barrier()

    @pl.loop(0, NRCHUNK)
    def _(ch):
        @pl.when(ch % NS == sid)
        def _():
            pltpu.sync_copy(acc.at[pl.ds(ch * RCHUNK, RCHUNK)],
                            out_hbm.at[cid, pl.ds(ch * RCHUNK, RCHUNK)])


def _matmul_body(x_ref, w_ref, o_ref):
    o_ref[...] = jnp.dot(x_ref[...], w_ref[...],
                         preferred_element_type=jnp.float32)


def _matmul(x, w):
    return pl.pallas_call(
        _matmul_body,
        grid=(GRID_TC,),
        in_specs=[
            pl.BlockSpec((ROWS_TC, D), lambda i: (i, 0)),
            pl.BlockSpec((D, D), lambda i: (0, 0)),
        ],
        out_specs=pl.BlockSpec((ROWS_TC, D), lambda i: (i, 0)),
        out_shape=jax.ShapeDtypeStruct((N, D), jnp.float32),
    )(x, w)


def _scale_body(degp_ref, xw_ref, y_ref, dis_ref):
    deg = degp_ref[0, :, 0:1] + degp_ref[1, :, 0:1]
    dis = lax.rsqrt(deg + 1.0)
    y_ref[...] = xw_ref[...] * dis
    dis_ref[...] = jnp.broadcast_to(dis, dis_ref.shape)


def _scale(deg_partials, xw):
    return pl.pallas_call(
        _scale_body,
        grid=(GRID_TC,),
        in_specs=[
            pl.BlockSpec((NC, ROWS_TC, D), lambda i: (0, i, 0)),
            pl.BlockSpec((ROWS_TC, D), lambda i: (i, 0)),
        ],
        out_specs=[
            pl.BlockSpec((ROWS_TC, D), lambda i: (i, 0)),
            pl.BlockSpec((ROWS_TC, 16), lambda i: (i, 0)),
        ],
        out_shape=[
            jax.ShapeDtypeStruct((N, D), jnp.float32),
            jax.ShapeDtypeStruct((N, 16), jnp.float32),
        ],
    )(deg_partials, xw)


def _mid_body(pa_ref, y_ref, dis_ref, b_ref, w_ref, y2_ref):
    agg = pa_ref[0] + pa_ref[1] + y_ref[...]
    dis = dis_ref[:, 0:1]
    h = jnp.maximum(agg * dis + b_ref[...], 0.0)
    y2_ref[...] = jnp.dot(h, w_ref[...],
                          preferred_element_type=jnp.float32) * dis


def _mid(partials, y, dis16, b, w):
    return pl.pallas_call(
        _mid_body,
        grid=(GRID_TC,),
        in_specs=[
            pl.BlockSpec((NC, ROWS_TC, D), lambda i: (0, i, 0)),
            pl.BlockSpec((ROWS_TC, D), lambda i: (i, 0)),
            pl.BlockSpec((ROWS_TC, 16), lambda i: (i, 0)),
            pl.BlockSpec((1, D), lambda i: (0, 0)),
            pl.BlockSpec((D, D), lambda i: (0, 0)),
        ],
        out_specs=pl.BlockSpec((ROWS_TC, D), lambda i: (i, 0)),
        out_shape=jax.ShapeDtypeStruct((N, D), jnp.float32),
    )(partials, y, dis16, b, w)


def _final_body(pa_ref, y_ref, dis_ref, b_ref, o_ref):
    agg = pa_ref[0] + pa_ref[1] + y_ref[...]
    o_ref[...] = agg * dis_ref[:, 0:1] + b_ref[...]


def _final(partials, y, dis16, b):
    return pl.pallas_call(
        _final_body,
        grid=(GRID_TC,),
        in_specs=[
            pl.BlockSpec((NC, ROWS_TC, D), lambda i: (0, i, 0)),
            pl.BlockSpec((ROWS_TC, D), lambda i: (i, 0)),
            pl.BlockSpec((ROWS_TC, 16), lambda i: (i, 0)),
            pl.BlockSpec((1, D), lambda i: (0, 0)),
        ],
        out_specs=pl.BlockSpec((ROWS_TC, D), lambda i: (i, 0)),
        out_shape=jax.ShapeDtypeStruct((N, D), jnp.float32),
    )(partials, y, dis16, b)


def kernel(x, edge_index, W1, b1, W2, b2):
    srcA = edge_index[0].astype(jnp.int32).reshape(NW, NBLK, BLK)
    dst = edge_index[1].astype(jnp.int32).reshape(NW, NBLK, BLK)
    ones128 = jnp.ones((BLK, D), jnp.float32)
    zeros128 = jnp.zeros((N, D), jnp.float32)
    b1r = b1.reshape(1, D)
    b2r = b2.reshape(1, D)

    deg_partials = _deg_kernel(dst, ones128, zeros128)
    xw1 = _matmul(x, W1)
    y1, dis16 = _scale(deg_partials, xw1)
    p1 = _agg_kernel(y1, srcA, dst, zeros128)
    y2 = _mid(p1, y1, dis16, b1r, W2)
    p2 = _agg_kernel(y2, srcA, dst, zeros128)
    return _final(p2, y2, dis16, b2r)


# speedup vs baseline: 1.2717x; 1.0037x over previous
import functools

import jax
import jax.numpy as jnp
from jax import lax
from jax.experimental import pallas as pl
from jax.experimental.pallas import tpu as pltpu
from jax.experimental.pallas import tpu_sc as plsc

N = 10000
D = 128
E = 320000

NC = 2
NS = 16
NW = NC * NS

EPW = E // NW
BLK = 125
NBLK = EPW // BLK
HBLK = NBLK // 2

KDEG = 5
RCHUNK = 400
NRCHUNK = N // RCHUNK

ROWS_TC = 1000
GRID_TC = N // ROWS_TC

_mesh = plsc.VectorSubcoreMesh(core_axis_name="c", subcore_axis_name="s")


@functools.partial(
    pl.kernel,
    mesh=_mesh,
    out_type=jax.ShapeDtypeStruct((NC, N, D), jnp.float32),
    scratch_types=[
        pltpu.VMEM((NBLK, BLK), jnp.int32),
        pltpu.VMEM((BLK, D), jnp.float32),
        pltpu.VMEM_SHARED((N, D), jnp.float32),
        pltpu.SemaphoreType.DMA,
    ],
)
def _deg_kernel(dst_hbm, ones_hbm, zeros_hbm, out_hbm, dst_v, ones_v, acc, sem):
    cid = lax.axis_index("c")
    sid = lax.axis_index("s")
    wid = sid * NC + cid

    @pl.loop(0, NRCHUNK)
    def _(ch):
        @pl.when(ch % NS == sid)
        def _():
            pltpu.sync_copy(zeros_hbm.at[pl.ds(ch * RCHUNK, RCHUNK)],
                            acc.at[pl.ds(ch * RCHUNK, RCHUNK)])

    pltpu.sync_copy(dst_hbm.at[wid], dst_v)
    pltpu.sync_copy(ones_hbm, ones_v)
    plsc.subcore_barrier()

    @pl.loop(0, NBLK // KDEG)
    def _(jb):
        for k in range(KDEG):
            pltpu.async_copy(ones_v, acc.at[dst_v.at[jb * KDEG + k]],
                             sem, add=True)
        for k in range(KDEG):
            pltpu.make_async_copy(ones_v, acc.at[dst_v.at[jb * KDEG + k]],
                                  sem).wait()
    for j in range((NBLK // KDEG) * KDEG, NBLK):
        pltpu.sync_copy(ones_v, acc.at[dst_v.at[j]], add=True)

    plsc.subcore_barrier()

    @pl.loop(0, NRCHUNK)
    def _(ch):
        @pl.when(ch % NS == sid)
        def _():
            pltpu.sync_copy(acc.at[pl.ds(ch * RCHUNK, RCHUNK)],
                            out_hbm.at[cid, pl.ds(ch * RCHUNK, RCHUNK)])


NBUF = 2
NFULL = (NBLK // NBUF) * NBUF


@functools.partial(
    pl.kernel,
    mesh=_mesh,
    out_type=jax.ShapeDtypeStruct((NC, N, D), jnp.float32),
    scratch_types=[
        pltpu.VMEM((NBLK, BLK), jnp.int32),
        pltpu.VMEM((HBLK, BLK), jnp.int32),
        pltpu.VMEM((BLK, D), jnp.float32),
        pltpu.VMEM((BLK, D), jnp.float32),
        pltpu.VMEM_SHARED((N, D), jnp.float32),
        pltpu.SemaphoreType.DMA,
        pltpu.SemaphoreType.DMA,
    ],
)
def _agg_kernel(y_hbm, src_hbm, dst_hbm, zeros_hbm, out_hbm,
                src_v, dst_v, rows0, rows1, acc, sem0, sem1):
    cid = lax.axis_index("c")
    sid = lax.axis_index("s")
    wid = sid * NC + cid
    rows = [rows0, rows1]
    sems = [sem0, sem1]

    @pl.loop(0, NRCHUNK)
    def _(ch):
        @pl.when(ch % NS == sid)
        def _():
            pltpu.sync_copy(zeros_hbm.at[pl.ds(ch * RCHUNK, RCHUNK)],
                            acc.at[pl.ds(ch * RCHUNK, RCHUNK)])

    pltpu.sync_copy(src_hbm.at[wid], src_v)
    pltpu.sync_copy(dst_hbm.at[wid, pl.ds(0, HBLK)], dst_v)
    plsc.subcore_barrier()

    for b in range(NBUF):
        pltpu.async_copy(y_hbm.at[src_v.at[b]], rows[b], sems[b])

    @pl.loop(0, NFULL, step=NBUF)
    def _(g):
        for b in range(NBUF):
            j = g + b
            if b == 0:
                @pl.when(g == HBLK)
                def _():
                    pltpu.sync_copy(dst_hbm.at[wid, pl.ds(HBLK, HBLK)], dst_v)
            pltpu.make_async_copy(y_hbm.at[src_v.at[j]],
                                  rows[b], sems[b]).wait()
            pltpu.sync_copy(rows[b], acc.at[dst_v.at[j % HBLK]], add=True)

            @pl.when(j + NBUF < NBLK)
            def _():
                pltpu.async_copy(y_hbm.at[src_v.at[j + NBUF]],
                                 rows[b], sems[b])

    for j in range(NFULL, NBLK):
        b = j % NBUF
        pltpu.make_async_copy(y_hbm.at[src_v.at[j]], rows[b], sems[b]).wait()
        pltpu.sync_copy(rows[b], acc.at[dst_v.at[j % HBLK]], add=True)

    plsc.subcore_barrier()

    @pl.loop(0, NRCHUNK)
    def _(ch):
        @pl.when(ch % NS == sid)
        def _():
            pltpu.sync_copy(acc.at[pl.ds(ch * RCHUNK, RCHUNK)],
                            out_hbm.at[cid, pl.ds(ch * RCHUNK, RCHUNK)])


def _scale_body(degp_ref, x_ref, w_ref, y_ref, dis_ref):
    deg = degp_ref[0, :, 0:1] + degp_ref[1, :, 0:1]
    dis = lax.rsqrt(deg + 1.0)
    y_ref[...] = jnp.dot(x_ref[...] * dis, w_ref[...],
                         preferred_element_type=jnp.float32)
    dis_ref[...] = jnp.broadcast_to(dis, dis_ref.shape)


def _scale(deg_partials, x, w):
    return pl.pallas_call(
        _scale_body,
        grid=(GRID_TC,),
        in_specs=[
            pl.BlockSpec((NC, ROWS_TC, D), lambda i: (0, i, 0)),
            pl.BlockSpec((ROWS_TC, D), lambda i: (i, 0)),
            pl.BlockSpec((D, D), lambda i: (0, 0)),
        ],
        out_specs=[
            pl.BlockSpec((ROWS_TC, D), lambda i: (i, 0)),
            pl.BlockSpec((ROWS_TC, 16), lambda i: (i, 0)),
        ],
        out_shape=[
            jax.ShapeDtypeStruct((N, D), jnp.float32),
            jax.ShapeDtypeStruct((N, 16), jnp.float32),
        ],
    )(deg_partials, x, w)


def _mid_body(pa_ref, y_ref, dis_ref, b_ref, w_ref, y2_ref):
    agg = pa_ref[0] + pa_ref[1] + y_ref[...]
    dis = dis_ref[:, 0:1]
    h = jnp.maximum(agg * dis + b_ref[...], 0.0)
    y2_ref[...] = jnp.dot(h, w_ref[...],
                          preferred_element_type=jnp.float32) * dis


def _mid(partials, y, dis16, b, w):
    return pl.pallas_call(
        _mid_body,
        grid=(GRID_TC,),
        in_specs=[
            pl.BlockSpec((NC, ROWS_TC, D), lambda i: (0, i, 0)),
            pl.BlockSpec((ROWS_TC, D), lambda i: (i, 0)),
            pl.BlockSpec((ROWS_TC, 16), lambda i: (i, 0)),
            pl.BlockSpec((1, D), lambda i: (0, 0)),
            pl.BlockSpec((D, D), lambda i: (0, 0)),
        ],
        out_specs=pl.BlockSpec((ROWS_TC, D), lambda i: (i, 0)),
        out_shape=jax.ShapeDtypeStruct((N, D), jnp.float32),
    )(partials, y, dis16, b, w)


def _final_body(pa_ref, y_ref, dis_ref, b_ref, o_ref):
    agg = pa_ref[0] + pa_ref[1] + y_ref[...]
    o_ref[...] = agg * dis_ref[:, 0:1] + b_ref[...]


def _final(partials, y, dis16, b):
    return pl.pallas_call(
        _final_body,
        grid=(GRID_TC,),
        in_specs=[
            pl.BlockSpec((NC, ROWS_TC, D), lambda i: (0, i, 0)),
            pl.BlockSpec((ROWS_TC, D), lambda i: (i, 0)),
            pl.BlockSpec((ROWS_TC, 16), lambda i: (i, 0)),
            pl.BlockSpec((1, D), lambda i: (0, 0)),
        ],
        out_specs=pl.BlockSpec((ROWS_TC, D), lambda i: (i, 0)),
        out_shape=jax.ShapeDtypeStruct((N, D), jnp.float32),
    )(partials, y, dis16, b)


def kernel(x, edge_index, W1, b1, W2, b2):
    srcA = edge_index[0].astype(jnp.int32).reshape(NW, NBLK, BLK)
    dst = edge_index[1].astype(jnp.int32).reshape(NW, NBLK, BLK)
    ones128 = jnp.ones((BLK, D), jnp.float32)
    zeros128 = jnp.zeros((N, D), jnp.float32)
    b1r = b1.reshape(1, D)
    b2r = b2.reshape(1, D)

    deg_partials = _deg_kernel(dst, ones128, zeros128)
    y1, dis16 = _scale(deg_partials, x, W1)
    p1 = _agg_kernel(y1, srcA, dst, zeros128)
    y2 = _mid(p1, y1, dis16, b1r, W2)
    p2 = _agg_kernel(y2, srcA, dst, zeros128)
    return _final(p2, y2, dis16, b2r)
